# Initial kernel scaffold; baseline (speedup 1.0000x reference)
import jax
import jax.numpy as jnp
from jax.experimental import pallas as pl


def kernel(hyperneigh, adj_src, adj_dst, adj_vals, emb_table, W1, b1):
    raise NotImplementedError("write your pallas kernel here")


# speedup vs baseline: 5.1944x; 5.1944x over previous
import functools

import jax
import jax.numpy as jnp
from jax import lax
from jax.experimental import pallas as pl
from jax.experimental.pallas import tpu as pltpu
from jax.experimental.pallas import tpu_sc as plsc

N = 10000
E = 320000
EMB = 128
HID = 100
D = 128

NC = 2
NS = 16
NW = NC * NS

GC = 80
NGC = N // GC

EC = 128
NEC = E // EC
KMAX = (NEC + NW - 1) // NW

ZR = 80
NZC = N // ZR

_mesh = plsc.VectorSubcoreMesh(core_axis_name="c", subcore_axis_name="s")


def _mm_body(x_ref, w_ref, o_ref):
    o_ref[...] = jnp.dot(x_ref[...], w_ref[...],
                         preferred_element_type=jnp.float32)


def _matmul(emb_table, w1p):
    return pl.pallas_call(
        _mm_body,
        grid=(10,),
        in_specs=[
            pl.BlockSpec((N // 10, EMB), lambda i: (i, 0)),
            pl.BlockSpec((EMB, D), lambda i: (0, 0)),
        ],
        out_specs=pl.BlockSpec((N // 10, D), lambda i: (i, 0)),
        out_shape=jax.ShapeDtypeStruct((N, D), jnp.float32),
    )(emb_table, w1p)


@functools.partial(
    pl.kernel,
    out_type=jax.ShapeDtypeStruct((N, D), jnp.float32),
    mesh=_mesh,
    scratch_types=[
        pltpu.VMEM((GC,), jnp.int32),
        pltpu.VMEM((GC, D), jnp.float32),
        pltpu.SemaphoreType.DMA,
    ],
)
def _sc_gather(tp_hbm, idx_hbm, out_hbm, idx_v, rows_v, sem):
    wid = lax.axis_index("s") * NC + lax.axis_index("c")

    def chunk(k, _):
        c = wid + k * NW

        @pl.when(c < NGC)
        def _():
            base = c * GC
            pltpu.sync_copy(idx_hbm.at[pl.ds(base, GC)], idx_v)
            pltpu.async_copy(tp_hbm.at[idx_v], rows_v, sem).wait()
            pltpu.sync_copy(rows_v, out_hbm.at[pl.ds(base, GC)])

        return 0

    lax.fori_loop(0, (NGC + NW - 1) // NW, chunk, 0)


@functools.partial(
    pl.kernel,
    out_type=jax.ShapeDtypeStruct((NC, N, D), jnp.float32),
    mesh=_mesh,
    scratch_types=[
        pltpu.VMEM_SHARED((N, D), jnp.float32),
        pltpu.VMEM((EC,), jnp.int32),
        pltpu.VMEM((EC,), jnp.int32),
        pltpu.VMEM((EC,), jnp.float32),
        pltpu.VMEM((EC, D), jnp.float32),
        pltpu.VMEM((ZR, D), jnp.float32),
        pltpu.SemaphoreType.DMA,
    ],
)
def _sc_edges(sup_hbm, src_hbm, dst_hbm, val_hbm, out_hbm,
              accum, src_v, dst_v, val_v, rows_v, zbuf, sem):
    cid = lax.axis_index("c")
    sid = lax.axis_index("s")
    wid = sid * NC + cid

    zv = jnp.zeros((16,), jnp.float32)

    def zrow(r, _):
        for j in range(D // 16):
            zbuf[r, pl.ds(16 * j, 16)] = zv
        return 0

    lax.fori_loop(0, ZR, zrow, 0)

    def zcopy(k, _):
        c = sid + k * NS

        @pl.when(c < NZC)
        def _():
            pltpu.sync_copy(zbuf, accum.at[pl.ds(c * ZR, ZR)])

        return 0

    lax.fori_loop(0, (NZC + NS - 1) // NS, zcopy, 0)
    plsc.subcore_barrier()

    def chunk(k, _):
        c = wid + k * NW

        @pl.when(c < NEC)
        def _():
            base = c * EC
            pltpu.sync_copy(src_hbm.at[pl.ds(base, EC)], src_v)
            pltpu.sync_copy(dst_hbm.at[pl.ds(base, EC)], dst_v)
            pltpu.sync_copy(val_hbm.at[pl.ds(base, EC)], val_v)
            pltpu.async_copy(sup_hbm.at[src_v], rows_v, sem).wait()

            def scale(g, _):
                vv = val_v[pl.ds(16 * g, 16)]
                for t in range(16):
                    v = vv[t]
                    i = 16 * g + t
                    for j in range(D // 16):
                        sl = pl.ds(16 * j, 16)
                        rows_v[i, sl] = rows_v[i, sl] * v
                return 0

            lax.fori_loop(0, EC // 16, scale, 0)
            pltpu.sync_copy(rows_v, accum.at[dst_v], add=True)

        return 0

    lax.fori_loop(0, KMAX, chunk, 0)
    plsc.subcore_barrier()

    def wcopy(k, _):
        c = sid + k * NS

        @pl.when(c < NZC)
        def _():
            base = c * ZR
            pltpu.sync_copy(accum.at[pl.ds(base, ZR)],
                            out_hbm.at[cid, pl.ds(base, ZR)])

        return 0

    lax.fori_loop(0, (NZC + NS - 1) // NS, wcopy, 0)


def _comb_body(a_ref, b_ref, bias_ref, o_ref):
    o_ref[...] = a_ref[...] + b_ref[...] + bias_ref[...]


def _combine(p0, p1, b1p):
    return pl.pallas_call(
        _comb_body,
        grid=(10,),
        in_specs=[
            pl.BlockSpec((N // 10, D), lambda i: (i, 0)),
            pl.BlockSpec((N // 10, D), lambda i: (i, 0)),
            pl.BlockSpec((1, D), lambda i: (0, 0)),
        ],
        out_specs=pl.BlockSpec((N // 10, D), lambda i: (i, 0)),
        out_shape=jax.ShapeDtypeStruct((N, D), jnp.float32),
    )(p0, p1, b1p)


def kernel(hyperneigh, adj_src, adj_dst, adj_vals, emb_table, W1, b1):
    w1p = jnp.pad(W1, ((0, 0), (0, D - HID)))
    b1p = jnp.pad(b1, (0, D - HID)).reshape(1, D)

    tp = _matmul(emb_table, w1p)
    support = _sc_gather(tp, hyperneigh.astype(jnp.int32))
    partials = _sc_edges(support,
                         adj_src.astype(jnp.int32),
                         adj_dst.astype(jnp.int32),
                         adj_vals)
    out = _combine(partials[0], partials[1], b1p)
    return out[:, :HID]
